# baseline (speedup 1.0000x reference)
import jax
import jax.numpy as jnp
from jax import lax
from jax.experimental import pallas as pl
from jax.experimental.pallas import tpu as pltpu
from jax.experimental.pallas import tpu_sc as plsc

B = 4096
L = 50
V = 100000
VH = V // 2
D = 64
NSC = 2
NSUB = 16
BPT = B // NSUB
NBUF = 8
NPASS = 4
RPH = BPT // NPASS
GT = 5


def _body(idx_hbm, val_hbm, emb_hbm, out_hbm,
          table_sh, idx_raw, idx_xf, val_slab, out_v,
          buf0, buf1, buf2, buf3, buf4, buf5, buf6, buf7,
          sem0, sem1, sem2, sem3, sem4, sem5, sem6, sem7):
    c = lax.axis_index("c")
    s = lax.axis_index("s")

    shard = VH // NSUB
    pltpu.sync_copy(
        emb_hbm.at[pl.ds(c * VH + s * shard, shard), :],
        table_sh.at[pl.ds(s * shard, shard), :])
    vbase = c * VH
    bufs = (buf0, buf1, buf2, buf3, buf4, buf5, buf6, buf7)
    sems = (sem0, sem1, sem2, sem3, sem4, sem5, sem6, sem7)

    for p in range(NPASS):
        rb = s * BPT + p * RPH
        pltpu.sync_copy(idx_hbm.at[pl.ds(rb, RPH), :], idx_raw)
        pltpu.sync_copy(val_hbm.at[pl.ds(rb, RPH), :], val_slab)

        def xform(r, carry):
            for off in (0, 16, 32, 34):
                iv = idx_raw[r, pl.ds(off, 16)]
                rel = iv - vbase
                owned = (rel >= 0) & (rel < VH)
                idx_xf[r, pl.ds(off, 16)] = jnp.clip(rel, 0, VH - 1)
                wv = val_slab[r, pl.ds(off, 16)]
                val_slab[r, pl.ds(off, 16)] = jnp.where(owned, wv, 0.0)
            return carry

        lax.fori_loop(0, RPH, xform, 0)
        if p == 0:
            plsc.subcore_barrier()

        for b in range(NBUF):
            pltpu.async_copy(
                table_sh.at[idx_xf.at[b]], bufs[b], sems[b])

        def step(g, carry):
            for b in range(NBUF):
                row = g * NBUF + b
                pltpu.make_async_copy(
                    table_sh.at[idx_xf.at[row]], bufs[b], sems[b]).wait()
                accs = [jnp.zeros((16,), jnp.float32) for _ in range(4)]
                rv = jnp.full((16,), row, jnp.int32)
                for l in range(L):
                    wv = plsc.load_gather(
                        val_slab, [rv, jnp.full((16,), l, jnp.int32)])
                    wvb = plsc.pack(
                        wv, wv, format=plsc.PackFormat.INTERLEAVED)
                    for h in range(2):
                        t = bufs[b][l, pl.ds(h * 32, 32)] * wvb
                        pa, pb = plsc.unpack(
                            t, format=plsc.PackFormat.INTERLEAVED)
                        accs[2 * h] = accs[2 * h] + pa
                        accs[2 * h + 1] = accs[2 * h + 1] + pb
                io2 = 2 * lax.iota(jnp.int32, 16)
                for h in range(2):
                    plsc.store_scatter(
                        out_v, [rv, h * 32 + io2], accs[2 * h])
                    plsc.store_scatter(
                        out_v, [rv, h * 32 + io2 + 1], accs[2 * h + 1])
                nxt = row + NBUF

                @pl.when(nxt < RPH)
                def _():
                    pltpu.async_copy(
                        table_sh.at[idx_xf.at[nxt]], bufs[b], sems[b])
            return carry

        lax.fori_loop(0, RPH // NBUF, step, 0)

        pltpu.sync_copy(out_v, out_hbm.at[c, pl.ds(rb, RPH), :])


@jax.jit
def _lookup(idx2d, val2d, emb_bf16):
    mesh = plsc.VectorSubcoreMesh(core_axis_name="c", subcore_axis_name="s")
    return pl.kernel(
        _body,
        out_type=jax.ShapeDtypeStruct((NSC, B, D), jnp.float32),
        mesh=mesh,
        compiler_params=pltpu.CompilerParams(
            needs_layout_passes=False, use_tc_tiling_on_sc=False),
        scratch_types=[
            pltpu.VMEM_SHARED((VH, D), jnp.bfloat16),
            pltpu.VMEM((RPH, L), jnp.int32),
            pltpu.VMEM((RPH, L), jnp.int32),
            pltpu.VMEM((RPH, L), jnp.float32),
            pltpu.VMEM((RPH, D), jnp.float32),
            pltpu.VMEM((L, D), jnp.bfloat16),
            pltpu.VMEM((L, D), jnp.bfloat16),
            pltpu.VMEM((L, D), jnp.bfloat16),
            pltpu.VMEM((L, D), jnp.bfloat16),
            pltpu.VMEM((L, D), jnp.bfloat16),
            pltpu.VMEM((L, D), jnp.bfloat16),
            pltpu.VMEM((L, D), jnp.bfloat16),
            pltpu.VMEM((L, D), jnp.bfloat16),
            pltpu.SemaphoreType.DMA,
            pltpu.SemaphoreType.DMA,
            pltpu.SemaphoreType.DMA,
            pltpu.SemaphoreType.DMA,
            pltpu.SemaphoreType.DMA,
            pltpu.SemaphoreType.DMA,
            pltpu.SemaphoreType.DMA,
            pltpu.SemaphoreType.DMA,
        ],
    )(idx2d, val2d, emb_bf16)


def _combine_body(p_ref, o_ref):
    o_ref[...] = p_ref[0] + p_ref[1]


@jax.jit
def _combine(partials):
    blk = 512
    return pl.pallas_call(
        _combine_body,
        grid=(B // blk,),
        in_specs=[pl.BlockSpec((NSC, blk, D), lambda i: (0, i, 0))],
        out_specs=pl.BlockSpec((blk, D), lambda i: (i, 0)),
        out_shape=jax.ShapeDtypeStruct((B, D), jnp.float32),
    )(partials)


def kernel(idx, val, embedding):
    partials = _lookup(idx.astype(jnp.int32), val.astype(jnp.float32),
                       embedding.astype(jnp.bfloat16))
    return _combine(partials)[:, None, :]


# speedup vs baseline: 1.1899x; 1.1899x over previous
import jax
import jax.numpy as jnp
from jax import lax
from jax.experimental import pallas as pl
from jax.experimental.pallas import tpu as pltpu
from jax.experimental.pallas import tpu_sc as plsc

B = 4096
L = 50
V = 100000
VH = V // 2
D = 64
NSC = 2
NSUB = 16
BPT = B // NSUB
NBUF = 2
NPASS = 4
RPH = BPT // NPASS
GT = 5


def _body(idx_hbm, val_hbm, emb_hbm, out_hbm,
          table_sh, idx_raw, idx_xf, val_slab, out_v,
          buf0, buf1, sem0, sem1):
    c = lax.axis_index("c")
    s = lax.axis_index("s")

    shard = VH // NSUB
    pltpu.sync_copy(
        emb_hbm.at[pl.ds(c * VH + s * shard, shard), :],
        table_sh.at[pl.ds(s * shard, shard), :])
    vbase = c * VH
    bufs = (buf0, buf1)
    sems = (sem0, sem1)

    for p in range(NPASS):
        rb = s * BPT + p * RPH
        pltpu.sync_copy(idx_hbm.at[pl.ds(rb, RPH), :], idx_raw)
        pltpu.sync_copy(val_hbm.at[pl.ds(rb, RPH), :], val_slab)

        def xform(r, carry):
            for off in (0, 16, 32, 34):
                iv = idx_raw[r, pl.ds(off, 16)]
                rel = iv - vbase
                owned = (rel >= 0) & (rel < VH)
                idx_xf[r, pl.ds(off, 16)] = jnp.clip(rel, 0, VH - 1)
                wv = val_slab[r, pl.ds(off, 16)]
                val_slab[r, pl.ds(off, 16)] = jnp.where(owned, wv, 0.0)
            return carry

        lax.fori_loop(0, RPH, xform, 0)
        if p == 0:
            plsc.subcore_barrier()

        for b in range(NBUF):
            pltpu.async_copy(
                table_sh.at[idx_xf.at[b]], bufs[b], sems[b])

        def step(g, carry):
            for b in range(NBUF):
                row = g * NBUF + b
                pltpu.make_async_copy(
                    table_sh.at[idx_xf.at[row]], bufs[b], sems[b]).wait()
                accs = [jnp.zeros((16,), jnp.float32) for _ in range(4)]
                rv = jnp.full((16,), row, jnp.int32)
                for l in range(L):
                    wv = plsc.load_gather(
                        val_slab, [rv, jnp.full((16,), l, jnp.int32)])
                    wvb = plsc.pack(
                        wv, wv, format=plsc.PackFormat.INTERLEAVED)
                    for h in range(2):
                        t = bufs[b][l, pl.ds(h * 32, 32)] * wvb
                        pa, pb = plsc.unpack(
                            t, format=plsc.PackFormat.INTERLEAVED)
                        accs[2 * h] = accs[2 * h] + pa
                        accs[2 * h + 1] = accs[2 * h + 1] + pb
                io2 = 2 * lax.iota(jnp.int32, 16)
                for h in range(2):
                    plsc.store_scatter(
                        out_v, [rv, h * 32 + io2], accs[2 * h])
                    plsc.store_scatter(
                        out_v, [rv, h * 32 + io2 + 1], accs[2 * h + 1])
                nxt = row + NBUF

                @pl.when(nxt < RPH)
                def _():
                    pltpu.async_copy(
                        table_sh.at[idx_xf.at[nxt]], bufs[b], sems[b])
            return carry

        lax.fori_loop(0, RPH // NBUF, step, 0)

        pltpu.sync_copy(out_v, out_hbm.at[c, pl.ds(rb, RPH), :])


@jax.jit
def _lookup(idx2d, val2d, emb_bf16):
    mesh = plsc.VectorSubcoreMesh(core_axis_name="c", subcore_axis_name="s")
    return pl.kernel(
        _body,
        out_type=jax.ShapeDtypeStruct((NSC, B, D), jnp.float32),
        mesh=mesh,
        compiler_params=pltpu.CompilerParams(
            needs_layout_passes=False, use_tc_tiling_on_sc=False),
        scratch_types=[
            pltpu.VMEM_SHARED((VH, D), jnp.bfloat16),
            pltpu.VMEM((RPH, L), jnp.int32),
            pltpu.VMEM((RPH, L), jnp.int32),
            pltpu.VMEM((RPH, L), jnp.float32),
            pltpu.VMEM((RPH, D), jnp.float32),
            pltpu.VMEM((L, D), jnp.bfloat16),
            pltpu.VMEM((L, D), jnp.bfloat16),
            pltpu.SemaphoreType.DMA,
            pltpu.SemaphoreType.DMA,
        ],
    )(idx2d, val2d, emb_bf16)


def _combine_body(p_ref, o_ref):
    o_ref[...] = p_ref[0] + p_ref[1]


@jax.jit
def _combine(partials):
    blk = 512
    return pl.pallas_call(
        _combine_body,
        grid=(B // blk,),
        in_specs=[pl.BlockSpec((NSC, blk, D), lambda i: (0, i, 0))],
        out_specs=pl.BlockSpec((blk, D), lambda i: (i, 0)),
        out_shape=jax.ShapeDtypeStruct((B, D), jnp.float32),
    )(partials)


def kernel(idx, val, embedding):
    partials = _lookup(idx.astype(jnp.int32), val.astype(jnp.float32),
                       embedding.astype(jnp.bfloat16))
    return _combine(partials)[:, None, :]
